# Initial kernel scaffold; baseline (speedup 1.0000x reference)
import jax
import jax.numpy as jnp
from jax.experimental import pallas as pl


def kernel(source_UV, source_VU, target_UV, target_VU, params):
    raise NotImplementedError("write your pallas kernel here")


# speedup vs baseline: 1.7046x; 1.7046x over previous
import functools

import jax
import jax.numpy as jnp
from jax.experimental import pallas as pl

F = 128
LEAK = 0.1


def _dot(a, b):
    return jnp.dot(a, b, preferred_element_type=jnp.float32)


def _leaky(x):
    return jnp.where(x >= 0, x, LEAK * x)


def _block_rows(n):
    return 200 if n % 200 == 0 else n


def _pre_body(ufea, share, vfea, wg1, wsg1, wg2, a_out, b_out):
    a_out[:, :F] = _dot(ufea[...], wg1[...])
    a_out[:, F:] = _dot(share[...], wsg1[...])
    b_out[...] = _dot(vfea[...], wg2[...])


def _vu1_body(vu, a_in, bias, wg3m, wsg2, y_out):
    h = _leaky(_dot(vu[...], a_in[...]) + bias[...])
    y_out[:, :F] = _dot(h[:, :F], wg3m[...])
    y_out[:, F:] = _dot(h[:, F:], wsg2[...])


def _uv_body(uv, x_in, bias, ufea, wuum, buum, wsum, bsum, wg4m, user_out, d_out):
    h = _leaky(_dot(uv[...], x_in[...]) + bias[...])
    item_ho = h[:, :F]
    u_mean = h[:, F:2 * F]
    sh2 = h[:, 2 * F:]
    slu = _dot(u_mean, wuum[:F, :]) + _dot(ufea[...], wuum[F:, :]) + buum[...]
    csm = _dot(sh2, wsum[:F, :]) + _dot(slu, wsum[F:, :]) + bsum[...]
    user_out[...] = csm + slu
    d_out[...] = _dot(item_ho, wg4m[...])


def _vu2_body(vu, d_in, bias, vfea, wium, bium, item_out):
    h = _leaky(_dot(vu[...], d_in[...]) + bias[...])
    item_out[...] = _dot(h, wium[:F, :]) + _dot(vfea[...], wium[F:, :]) + bium[...]


def _const_spec(shape):
    return pl.BlockSpec(shape, lambda i: (0,) * len(shape))


def _row_spec(bm, width):
    return pl.BlockSpec((bm, width), lambda i: (i, 0))


def _row1(b):
    return b.reshape(1, -1)


def _domain(UV, VU, ufea, vfea, share, spec, cgc1, cgc2, cum):
    nu = ufea.shape[0]
    ni = vfea.shape[0]
    f32 = jnp.float32

    a_mat, b_mat = pl.pallas_call(
        _pre_body,
        out_shape=[jax.ShapeDtypeStruct((nu, 2 * F), f32),
                   jax.ShapeDtypeStruct((ni, F), f32)],
    )(ufea, share, vfea, spec["gc1"]["W"], cgc1["W"], spec["gc2"]["W"])

    bm = _block_rows(ni)
    bias1 = jnp.concatenate([_row1(spec["gc1"]["b"]), _row1(cgc1["b"])], axis=1)
    y_mat = pl.pallas_call(
        _vu1_body,
        grid=(ni // bm,),
        in_specs=[_row_spec(bm, nu), _const_spec((nu, 2 * F)),
                  _const_spec((1, 2 * F)), _const_spec((F, F)),
                  _const_spec((F, F))],
        out_specs=_row_spec(bm, 2 * F),
        out_shape=jax.ShapeDtypeStruct((ni, 2 * F), f32),
    )(VU, a_mat, bias1, spec["gc3m"]["W"], cgc2["W"])

    x_mat = jnp.concatenate([b_mat, y_mat], axis=1)
    bm = _block_rows(nu)
    bias2 = jnp.concatenate([_row1(spec["gc2"]["b"]), _row1(spec["gc3m"]["b"]),
                             _row1(cgc2["b"])], axis=1)
    user_out, d_mat = pl.pallas_call(
        _uv_body,
        grid=(nu // bm,),
        in_specs=[_row_spec(bm, ni), _const_spec((ni, 3 * F)),
                  _const_spec((1, 3 * F)), _row_spec(bm, F),
                  _const_spec((2 * F, F)), _const_spec((1, F)),
                  _const_spec((2 * F, F)), _const_spec((1, F)),
                  _const_spec((F, F))],
        out_specs=[_row_spec(bm, F), _row_spec(bm, F)],
        out_shape=[jax.ShapeDtypeStruct((nu, F), f32),
                   jax.ShapeDtypeStruct((nu, F), f32)],
    )(UV, x_mat, bias2, ufea, spec["uum"]["W"], _row1(spec["uum"]["b"]),
      cum["W"], _row1(cum["b"]), spec["gc4m"]["W"])

    bm = _block_rows(ni)
    item_out = pl.pallas_call(
        _vu2_body,
        grid=(ni // bm,),
        in_specs=[_row_spec(bm, nu), _const_spec((nu, F)),
                  _const_spec((1, F)), _row_spec(bm, F),
                  _const_spec((2 * F, F)), _const_spec((1, F))],
        out_specs=_row_spec(bm, F),
        out_shape=jax.ShapeDtypeStruct((ni, F), f32),
    )(VU, d_mat, _row1(spec["gc4m"]["b"]), vfea,
      spec["ium"]["W"], _row1(spec["ium"]["b"]))

    return user_out, item_out


@functools.partial(jax.jit)
def kernel(source_UV, source_VU, target_UV, target_VU, params):
    cond = params["cond"]
    s_user, s_item = _domain(source_UV, source_VU,
                             params["src_user_emb"], params["src_item_emb"],
                             params["src_user_share"], params["src_specific"],
                             cond["s_gc1"], cond["s_gc2"], cond["s_um"])
    t_user, t_item = _domain(target_UV, target_VU,
                             params["tgt_user_emb"], params["tgt_item_emb"],
                             params["tgt_user_share"], params["tgt_specific"],
                             cond["t_gc1"], cond["t_gc2"], cond["t_um"])
    return s_user, s_item, t_user, t_item


# speedup vs baseline: 1.7499x; 1.0266x over previous
import functools

import jax
import jax.numpy as jnp
from jax.experimental import pallas as pl

F = 128
LEAK = 0.1


def _dot(a, b):
    return jnp.dot(a, b, preferred_element_type=jnp.float32)


def _leaky(x):
    return jnp.where(x >= 0, x, LEAK * x)


def _block_rows(n):
    return 200 if n % 200 == 0 else n


def _pre_body(su, ss, sv, tu, ts, tv, w1s, w2s, w3s, w1t, w2t, w3t,
              a_s, b_s, a_t, b_t):
    a_s[:, :F] = _dot(su[...], w1s[...])
    a_s[:, F:] = _dot(ss[...], w2s[...])
    b_s[...] = _dot(sv[...], w3s[...])
    a_t[:, :F] = _dot(tu[...], w1t[...])
    a_t[:, F:] = _dot(ts[...], w2t[...])
    b_t[...] = _dot(tv[...], w3t[...])


def _vu1_body(vu, a_in, bias, wg3m, wsg2, y_out):
    h = _leaky(_dot(vu[...], a_in[...]) + bias[...])
    y_out[:, :F] = _dot(h[:, :F], wg3m[...])
    y_out[:, F:] = _dot(h[:, F:], wsg2[...])


def _uv_body(uv, b_in, y_in, bias_b, bias_y, ufea, wuum, buum, wsum, bsum,
             wg4m, user_out, d_out):
    blk = uv[...]
    item_ho = _leaky(_dot(blk, b_in[...]) + bias_b[...])
    h = _leaky(_dot(blk, y_in[...]) + bias_y[...])
    u_mean = h[:, :F]
    sh2 = h[:, F:]
    slu = _dot(u_mean, wuum[:F, :]) + _dot(ufea[...], wuum[F:, :]) + buum[...]
    csm = _dot(sh2, wsum[:F, :]) + _dot(slu, wsum[F:, :]) + bsum[...]
    user_out[...] = csm + slu
    d_out[...] = _dot(item_ho, wg4m[...])


def _vu2_body(vu, d_in, bias, vfea, wium, bium, item_out):
    h = _leaky(_dot(vu[...], d_in[...]) + bias[...])
    item_out[...] = _dot(h, wium[:F, :]) + _dot(vfea[...], wium[F:, :]) + bium[...]


def _const_spec(shape):
    return pl.BlockSpec(shape, lambda i: (0,) * len(shape))


def _row_spec(bm, width):
    return pl.BlockSpec((bm, width), lambda i: (i, 0))


def _row1(b):
    return b.reshape(1, -1)


def _domain(UV, VU, a_mat, b_mat, ufea, vfea, spec, cgc1, cgc2, cum):
    nu = ufea.shape[0]
    ni = vfea.shape[0]
    f32 = jnp.float32

    bm = _block_rows(ni)
    bias1 = jnp.concatenate([_row1(spec["gc1"]["b"]), _row1(cgc1["b"])], axis=1)
    y_mat = pl.pallas_call(
        _vu1_body,
        grid=(ni // bm,),
        in_specs=[_row_spec(bm, nu), _const_spec((nu, 2 * F)),
                  _const_spec((1, 2 * F)), _const_spec((F, F)),
                  _const_spec((F, F))],
        out_specs=_row_spec(bm, 2 * F),
        out_shape=jax.ShapeDtypeStruct((ni, 2 * F), f32),
    )(VU, a_mat, bias1, spec["gc3m"]["W"], cgc2["W"])

    bm = _block_rows(nu)
    bias_y = jnp.concatenate([_row1(spec["gc3m"]["b"]), _row1(cgc2["b"])], axis=1)
    user_out, d_mat = pl.pallas_call(
        _uv_body,
        grid=(nu // bm,),
        in_specs=[_row_spec(bm, ni), _const_spec((ni, F)),
                  _const_spec((ni, 2 * F)), _const_spec((1, F)),
                  _const_spec((1, 2 * F)), _row_spec(bm, F),
                  _const_spec((2 * F, F)), _const_spec((1, F)),
                  _const_spec((2 * F, F)), _const_spec((1, F)),
                  _const_spec((F, F))],
        out_specs=[_row_spec(bm, F), _row_spec(bm, F)],
        out_shape=[jax.ShapeDtypeStruct((nu, F), f32),
                   jax.ShapeDtypeStruct((nu, F), f32)],
    )(UV, b_mat, y_mat, _row1(spec["gc2"]["b"]), bias_y, ufea,
      spec["uum"]["W"], _row1(spec["uum"]["b"]),
      cum["W"], _row1(cum["b"]), spec["gc4m"]["W"])

    bm = _block_rows(ni)
    item_out = pl.pallas_call(
        _vu2_body,
        grid=(ni // bm,),
        in_specs=[_row_spec(bm, nu), _const_spec((nu, F)),
                  _const_spec((1, F)), _row_spec(bm, F),
                  _const_spec((2 * F, F)), _const_spec((1, F))],
        out_specs=_row_spec(bm, F),
        out_shape=jax.ShapeDtypeStruct((ni, F), f32),
    )(VU, d_mat, _row1(spec["gc4m"]["b"]), vfea,
      spec["ium"]["W"], _row1(spec["ium"]["b"]))

    return user_out, item_out


@functools.partial(jax.jit)
def kernel(source_UV, source_VU, target_UV, target_VU, params):
    cond = params["cond"]
    f32 = jnp.float32
    s_spec, t_spec = params["src_specific"], params["tgt_specific"]
    su, sv = params["src_user_emb"], params["src_item_emb"]
    tu, tv = params["tgt_user_emb"], params["tgt_item_emb"]

    n = su.shape[0]
    bm = 1000 if n % 1000 == 0 else n
    a_s, b_s, a_t, b_t = pl.pallas_call(
        _pre_body,
        grid=(n // bm,),
        in_specs=[_row_spec(bm, F)] * 6 + [_const_spec((F, F))] * 6,
        out_specs=[_row_spec(bm, 2 * F), _row_spec(bm, F),
                   _row_spec(bm, 2 * F), _row_spec(bm, F)],
        out_shape=[jax.ShapeDtypeStruct((su.shape[0], 2 * F), f32),
                   jax.ShapeDtypeStruct((sv.shape[0], F), f32),
                   jax.ShapeDtypeStruct((tu.shape[0], 2 * F), f32),
                   jax.ShapeDtypeStruct((tv.shape[0], F), f32)],
    )(su, params["src_user_share"], sv, tu, params["tgt_user_share"], tv,
      s_spec["gc1"]["W"], cond["s_gc1"]["W"], s_spec["gc2"]["W"],
      t_spec["gc1"]["W"], cond["t_gc1"]["W"], t_spec["gc2"]["W"])

    s_user, s_item = _domain(source_UV, source_VU, a_s, b_s, su, sv,
                             s_spec, cond["s_gc1"], cond["s_gc2"], cond["s_um"])
    t_user, t_item = _domain(target_UV, target_VU, a_t, b_t, tu, tv,
                             t_spec, cond["t_gc1"], cond["t_gc2"], cond["t_um"])
    return s_user, s_item, t_user, t_item


# speedup vs baseline: 1.7821x; 1.0184x over previous
import functools

import jax
import jax.numpy as jnp
from jax.experimental import pallas as pl

F = 128
LEAK = 0.1


def _dot(a, b):
    return jnp.dot(a, b, preferred_element_type=jnp.float32)


def _leaky(x):
    return jnp.where(x >= 0, x, LEAK * x)


def _block_rows(n, bm):
    return bm if n % bm == 0 else n


def _pre_body(su, ss, sv, tu, ts, tv, w1s, w2s, w3s, w1t, w2t, w3t,
              a_s, b_s, a_t, b_t):
    a_s[:, :F] = _dot(su[...], w1s[...])
    a_s[:, F:] = _dot(ss[...], w2s[...])
    b_s[...] = _dot(sv[...], w3s[...])
    a_t[:, :F] = _dot(tu[...], w1t[...])
    a_t[:, F:] = _dot(ts[...], w2t[...])
    b_t[...] = _dot(tv[...], w3t[...])


def _vu1_body(vu, a_in, bias, wg3m, wsg2, y_out):
    h = _leaky(_dot(vu[...], a_in[...]) + bias[...])
    y_out[:, :F] = _dot(h[:, :F], wg3m[...])
    y_out[:, F:] = _dot(h[:, F:], wsg2[...])


def _uv_body(uv, b_in, y_in, bias_b, bias_y, ufea, wuum, buum, wsum, bsum,
             wg4m, user_out, d_out):
    blk = uv[...]
    item_ho = _leaky(_dot(blk, b_in[...]) + bias_b[...])
    h = _leaky(_dot(blk, y_in[...]) + bias_y[...])
    u_mean = h[:, :F]
    sh2 = h[:, F:]
    slu = _dot(u_mean, wuum[:F, :]) + _dot(ufea[...], wuum[F:, :]) + buum[...]
    csm = _dot(sh2, wsum[:F, :]) + _dot(slu, wsum[F:, :]) + bsum[...]
    user_out[...] = csm + slu
    d_out[...] = _dot(item_ho, wg4m[...])


def _vu2_body(vu, d_in, bias, vfea, wium, bium, item_out):
    h = _leaky(_dot(vu[...], d_in[...]) + bias[...])
    item_out[...] = _dot(h, wium[:F, :]) + _dot(vfea[...], wium[F:, :]) + bium[...]


def _const_spec(shape):
    return pl.BlockSpec(shape, lambda i: (0,) * len(shape))


def _row_spec(bm, width):
    return pl.BlockSpec((bm, width), lambda i: (i, 0))


def _row1(b):
    return b.reshape(1, -1)


def _domain(UV, VU, a_mat, b_mat, ufea, vfea, spec, cgc1, cgc2, cum):
    nu = ufea.shape[0]
    ni = vfea.shape[0]
    f32 = jnp.float32

    bm = _block_rows(ni, 400)
    bias1 = jnp.concatenate([_row1(spec["gc1"]["b"]), _row1(cgc1["b"])], axis=1)
    y_mat = pl.pallas_call(
        _vu1_body,
        grid=(ni // bm,),
        in_specs=[_row_spec(bm, nu), _const_spec((nu, 2 * F)),
                  _const_spec((1, 2 * F)), _const_spec((F, F)),
                  _const_spec((F, F))],
        out_specs=_row_spec(bm, 2 * F),
        out_shape=jax.ShapeDtypeStruct((ni, 2 * F), f32),
    )(VU, a_mat, bias1, spec["gc3m"]["W"], cgc2["W"])

    bm = _block_rows(nu, 200)
    bias_y = jnp.concatenate([_row1(spec["gc3m"]["b"]), _row1(cgc2["b"])], axis=1)
    user_out, d_mat = pl.pallas_call(
        _uv_body,
        grid=(nu // bm,),
        in_specs=[_row_spec(bm, ni), _const_spec((ni, F)),
                  _const_spec((ni, 2 * F)), _const_spec((1, F)),
                  _const_spec((1, 2 * F)), _row_spec(bm, F),
                  _const_spec((2 * F, F)), _const_spec((1, F)),
                  _const_spec((2 * F, F)), _const_spec((1, F)),
                  _const_spec((F, F))],
        out_specs=[_row_spec(bm, F), _row_spec(bm, F)],
        out_shape=[jax.ShapeDtypeStruct((nu, F), f32),
                   jax.ShapeDtypeStruct((nu, F), f32)],
    )(UV, b_mat, y_mat, _row1(spec["gc2"]["b"]), bias_y, ufea,
      spec["uum"]["W"], _row1(spec["uum"]["b"]),
      cum["W"], _row1(cum["b"]), spec["gc4m"]["W"])

    bm = _block_rows(ni, 400)
    item_out = pl.pallas_call(
        _vu2_body,
        grid=(ni // bm,),
        in_specs=[_row_spec(bm, nu), _const_spec((nu, F)),
                  _const_spec((1, F)), _row_spec(bm, F),
                  _const_spec((2 * F, F)), _const_spec((1, F))],
        out_specs=_row_spec(bm, F),
        out_shape=jax.ShapeDtypeStruct((ni, F), f32),
    )(VU, d_mat, _row1(spec["gc4m"]["b"]), vfea,
      spec["ium"]["W"], _row1(spec["ium"]["b"]))

    return user_out, item_out


@functools.partial(jax.jit)
def kernel(source_UV, source_VU, target_UV, target_VU, params):
    cond = params["cond"]
    f32 = jnp.float32
    s_spec, t_spec = params["src_specific"], params["tgt_specific"]
    su, sv = params["src_user_emb"], params["src_item_emb"]
    tu, tv = params["tgt_user_emb"], params["tgt_item_emb"]

    n = su.shape[0]
    bm = 1000 if n % 1000 == 0 else n
    a_s, b_s, a_t, b_t = pl.pallas_call(
        _pre_body,
        grid=(n // bm,),
        in_specs=[_row_spec(bm, F)] * 6 + [_const_spec((F, F))] * 6,
        out_specs=[_row_spec(bm, 2 * F), _row_spec(bm, F),
                   _row_spec(bm, 2 * F), _row_spec(bm, F)],
        out_shape=[jax.ShapeDtypeStruct((su.shape[0], 2 * F), f32),
                   jax.ShapeDtypeStruct((sv.shape[0], F), f32),
                   jax.ShapeDtypeStruct((tu.shape[0], 2 * F), f32),
                   jax.ShapeDtypeStruct((tv.shape[0], F), f32)],
    )(su, params["src_user_share"], sv, tu, params["tgt_user_share"], tv,
      s_spec["gc1"]["W"], cond["s_gc1"]["W"], s_spec["gc2"]["W"],
      t_spec["gc1"]["W"], cond["t_gc1"]["W"], t_spec["gc2"]["W"])

    s_user, s_item = _domain(source_UV, source_VU, a_s, b_s, su, sv,
                             s_spec, cond["s_gc1"], cond["s_gc2"], cond["s_um"])
    t_user, t_item = _domain(target_UV, target_VU, a_t, b_t, tu, tv,
                             t_spec, cond["t_gc1"], cond["t_gc2"], cond["t_um"])
    return s_user, s_item, t_user, t_item


# speedup vs baseline: 1.8946x; 1.0631x over previous
import functools

import jax
import jax.numpy as jnp
from jax.experimental import pallas as pl

F = 128
LEAK = 0.1


def _dot(a, b):
    return jnp.dot(a, b, preferred_element_type=jnp.float32)


def _dotm(a, x):
    return jax.lax.dot_general(a, x, (((1,), (0,)), ((), ())),
                               preferred_element_type=jnp.float32)


def _bf(x):
    return x.astype(jnp.bfloat16)


def _leaky(x):
    return jnp.where(x >= 0, x, LEAK * x)


def _block_rows(n, bm):
    return bm if n % bm == 0 else n


def _pre_body(su, ss, sv, tu, ts, tv, w1s, w2s, w3s, w1t, w2t, w3t,
              a_s, b_s, a_t, b_t):
    a_s[:, :F] = _bf(_dot(su[...], w1s[...]))
    a_s[:, F:] = _bf(_dot(ss[...], w2s[...]))
    b_s[...] = _bf(_dot(sv[...], w3s[...]))
    a_t[:, :F] = _bf(_dot(tu[...], w1t[...]))
    a_t[:, F:] = _bf(_dot(ts[...], w2t[...]))
    b_t[...] = _bf(_dot(tv[...], w3t[...]))


def _vu1_body(vu, a_in, bias, wg3m, wsg2, y_out):
    h = _leaky(_dotm(vu[...], a_in[...]) + bias[...])
    y_out[:, :F] = _bf(_dot(h[:, :F], wg3m[...]))
    y_out[:, F:] = _bf(_dot(h[:, F:], wsg2[...]))


def _uv_body(uv, b_in, y_in, bias_b, bias_y, ufea, wuum, buum, wsum, bsum,
             wg4m, user_out, d_out):
    blk = uv[...]
    item_ho = _leaky(_dotm(blk, b_in[...]) + bias_b[...])
    h = _leaky(_dotm(blk, y_in[...]) + bias_y[...])
    u_mean = h[:, :F]
    sh2 = h[:, F:]
    slu = _dot(u_mean, wuum[:F, :]) + _dot(ufea[...], wuum[F:, :]) + buum[...]
    csm = _dot(sh2, wsum[:F, :]) + _dot(slu, wsum[F:, :]) + bsum[...]
    user_out[...] = csm + slu
    d_out[...] = _bf(_dot(item_ho, wg4m[...]))


def _vu2_body(vu, d_in, bias, vfea, wium, bium, item_out):
    h = _leaky(_dotm(vu[...], d_in[...]) + bias[...])
    item_out[...] = _dot(h, wium[:F, :]) + _dot(vfea[...], wium[F:, :]) + bium[...]


def _const_spec(shape):
    return pl.BlockSpec(shape, lambda i: (0,) * len(shape))


def _row_spec(bm, width):
    return pl.BlockSpec((bm, width), lambda i: (i, 0))


def _row1(b):
    return b.reshape(1, -1)


def _domain(UV, VU, a_mat, b_mat, ufea, vfea, spec, cgc1, cgc2, cum):
    nu = ufea.shape[0]
    ni = vfea.shape[0]
    f32 = jnp.float32

    bm = _block_rows(ni, 400)
    bias1 = jnp.concatenate([_row1(spec["gc1"]["b"]), _row1(cgc1["b"])], axis=1)
    y_mat = pl.pallas_call(
        _vu1_body,
        grid=(ni // bm,),
        in_specs=[_row_spec(bm, nu), _const_spec((nu, 2 * F)),
                  _const_spec((1, 2 * F)), _const_spec((F, F)),
                  _const_spec((F, F))],
        out_specs=_row_spec(bm, 2 * F),
        out_shape=jax.ShapeDtypeStruct((ni, 2 * F), jnp.bfloat16),
    )(VU, a_mat, bias1, spec["gc3m"]["W"], cgc2["W"])

    bm = _block_rows(nu, 400)
    bias_y = jnp.concatenate([_row1(spec["gc3m"]["b"]), _row1(cgc2["b"])], axis=1)
    user_out, d_mat = pl.pallas_call(
        _uv_body,
        grid=(nu // bm,),
        in_specs=[_row_spec(bm, ni), _const_spec((ni, F)),
                  _const_spec((ni, 2 * F)), _const_spec((1, F)),
                  _const_spec((1, 2 * F)), _row_spec(bm, F),
                  _const_spec((2 * F, F)), _const_spec((1, F)),
                  _const_spec((2 * F, F)), _const_spec((1, F)),
                  _const_spec((F, F))],
        out_specs=[_row_spec(bm, F), _row_spec(bm, F)],
        out_shape=[jax.ShapeDtypeStruct((nu, F), f32),
                   jax.ShapeDtypeStruct((nu, F), jnp.bfloat16)],
    )(UV, b_mat, y_mat, _row1(spec["gc2"]["b"]), bias_y, ufea,
      spec["uum"]["W"], _row1(spec["uum"]["b"]),
      cum["W"], _row1(cum["b"]), spec["gc4m"]["W"])

    bm = _block_rows(ni, 400)
    item_out = pl.pallas_call(
        _vu2_body,
        grid=(ni // bm,),
        in_specs=[_row_spec(bm, nu), _const_spec((nu, F)),
                  _const_spec((1, F)), _row_spec(bm, F),
                  _const_spec((2 * F, F)), _const_spec((1, F))],
        out_specs=_row_spec(bm, F),
        out_shape=jax.ShapeDtypeStruct((ni, F), f32),
    )(VU, d_mat, _row1(spec["gc4m"]["b"]), vfea,
      spec["ium"]["W"], _row1(spec["ium"]["b"]))

    return user_out, item_out


@functools.partial(jax.jit)
def kernel(source_UV, source_VU, target_UV, target_VU, params):
    cond = params["cond"]
    f32 = jnp.float32
    s_spec, t_spec = params["src_specific"], params["tgt_specific"]
    su, sv = params["src_user_emb"], params["src_item_emb"]
    tu, tv = params["tgt_user_emb"], params["tgt_item_emb"]

    n = su.shape[0]
    bm = 1000 if n % 1000 == 0 else n
    a_s, b_s, a_t, b_t = pl.pallas_call(
        _pre_body,
        grid=(n // bm,),
        in_specs=[_row_spec(bm, F)] * 6 + [_const_spec((F, F))] * 6,
        out_specs=[_row_spec(bm, 2 * F), _row_spec(bm, F),
                   _row_spec(bm, 2 * F), _row_spec(bm, F)],
        out_shape=[jax.ShapeDtypeStruct((su.shape[0], 2 * F), jnp.bfloat16),
                   jax.ShapeDtypeStruct((sv.shape[0], F), jnp.bfloat16),
                   jax.ShapeDtypeStruct((tu.shape[0], 2 * F), jnp.bfloat16),
                   jax.ShapeDtypeStruct((tv.shape[0], F), jnp.bfloat16)],
    )(su, params["src_user_share"], sv, tu, params["tgt_user_share"], tv,
      s_spec["gc1"]["W"], cond["s_gc1"]["W"], s_spec["gc2"]["W"],
      t_spec["gc1"]["W"], cond["t_gc1"]["W"], t_spec["gc2"]["W"])

    s_user, s_item = _domain(source_UV, source_VU, a_s, b_s, su, sv,
                             s_spec, cond["s_gc1"], cond["s_gc2"], cond["s_um"])
    t_user, t_item = _domain(target_UV, target_VU, a_t, b_t, tu, tv,
                             t_spec, cond["t_gc1"], cond["t_gc2"], cond["t_um"])
    return s_user, s_item, t_user, t_item
